# baseline (speedup 1.0000x reference)
import functools

import jax
import jax.numpy as jnp
from jax import lax
from jax.experimental import pallas as pl
from jax.experimental.pallas import tpu as pltpu
from jax.experimental.pallas import tpu_sc as plsc

_L = 16


@functools.lru_cache(maxsize=None)
def _build_gather(B, D, NC, NS):
    D8 = 8 * D
    NW = NC * NS
    b_per_w = B // NW
    K = 32
    n_chunks = b_per_w // K
    mesh = plsc.VectorSubcoreMesh(core_axis_name="c", subcore_axis_name="s")

    @functools.partial(
        pl.kernel,
        mesh=mesh,
        out_type=jax.ShapeDtypeStruct((B // 2, 2 * D), jnp.float32),
        scratch_types=[
            pltpu.VMEM((b_per_w,), jnp.int32),
            pltpu.VMEM((b_per_w,), jnp.int32),
            pltpu.VMEM((2, K, D8), jnp.float32),
            pltpu.VMEM((2, K // 2, 2 * D), jnp.float32),
            pltpu.SemaphoreType.DMA((2,)),
            pltpu.SemaphoreType.DMA((2,)),
        ],
    )
    def k(idx_hbm, table_hbm, out_hbm, idx_v, tidx_v, tiles_v, out_v,
          gsem, wsem):
        wid = lax.axis_index("s") * NC + lax.axis_index("c")
        base = wid * b_per_w
        pltpu.sync_copy(idx_hbm.at[pl.ds(base, b_per_w)], idx_v)
        for c in range(b_per_w // _L):
            v = idx_v[pl.ds(c * _L, _L)]
            tidx_v[pl.ds(c * _L, _L)] = lax.shift_right_logical(v, 3)

        def gather(i, buf):
            return pltpu.make_async_copy(
                table_hbm.at[tidx_v.at[pl.ds(i * K, K)]],
                tiles_v.at[buf],
                gsem.at[buf],
            )

        def writeback(i, buf):
            return pltpu.make_async_copy(
                out_v.at[buf],
                out_hbm.at[pl.ds(pl.multiple_of((base + i * K) // 2, 8), K // 2)],
                wsem.at[buf],
            )

        def extract(i, buf):
            for g in range(K // _L):
                ids = idx_v[pl.ds(i * K + g * _L, _L)]
                for l in range(_L):
                    r = ids[l] & 7
                    j = g * _L + l
                    for c in range(D // _L):
                        out_v[buf, j // 2, pl.ds((j % 2) * D + c * _L, _L)] = (
                            tiles_v[buf, j, pl.ds(r * D + c * _L, _L)]
                        )

        gather(0, 0).start()
        for i in range(n_chunks):
            buf = i % 2
            if i + 1 < n_chunks:
                gather(i + 1, 1 - buf).start()
            gather(i, buf).wait()
            if i >= 2:
                writeback(i - 2, buf).wait()
            extract(i, buf)
            writeback(i, buf).start()
        for i in (n_chunks - 2, n_chunks - 1):
            if i >= 0:
                writeback(i, i % 2).wait()

    return k


def kernel(node_ids, emb_weight):
    node_ids = node_ids.astype(jnp.int32)
    (B,) = node_ids.shape
    V, D = emb_weight.shape
    table8 = emb_weight.reshape(V // 8, 8 * D)
    info = plsc.get_sparse_core_info()
    k = _build_gather(B, D, info.num_cores, info.num_subcores)
    out2 = k(node_ids, table8)
    return out2.reshape(B, D)


# speedup vs baseline: 1.6841x; 1.6841x over previous
import functools

import jax
import jax.numpy as jnp
from jax import lax
from jax.experimental import pallas as pl
from jax.experimental.pallas import tpu as pltpu
from jax.experimental.pallas import tpu_sc as plsc

_L = 16


@functools.lru_cache(maxsize=None)
def _build_gather(B, D, NC, NS):
    NW = NC * NS
    b_per_w = B // NW
    K = 32
    n_chunks = b_per_w // K
    mesh = plsc.VectorSubcoreMesh(core_axis_name="c", subcore_axis_name="s")

    @functools.partial(
        pl.kernel,
        mesh=mesh,
        out_type=jax.ShapeDtypeStruct((B // 2, 2 * D), jnp.float32),
        scratch_types=[
            pltpu.VMEM((b_per_w,), jnp.int32),
            pltpu.VMEM((2, K, 8, D), jnp.float32),
            pltpu.VMEM((2, K // 2, 2 * D), jnp.float32),
            pltpu.SemaphoreType.DMA((2,)),
            pltpu.SemaphoreType.DMA((2,)),
        ],
    )
    def k(idx_hbm, table_hbm, out_hbm, idx_v, tiles_v, out_v, gsem, wsem):
        wid = lax.axis_index("s") * NC + lax.axis_index("c")
        base = wid * b_per_w
        pltpu.sync_copy(idx_hbm.at[pl.ds(base, b_per_w)], idx_v)

        def issue(i):
            buf = i % 2
            for g in range(K // _L):
                ids = idx_v[pl.ds(i * K + g * _L, _L)]
                for l in range(_L):
                    t8 = pl.multiple_of((ids[l] >> 3) << 3, 8)
                    pltpu.async_copy(
                        table_hbm.at[pl.ds(t8, 8)],
                        tiles_v.at[buf, g * _L + l],
                        gsem.at[buf],
                    )

        def drain(i):
            buf = i % 2
            for j in range(K):
                pltpu.make_async_copy(
                    table_hbm.at[pl.ds(0, 8)],
                    tiles_v.at[buf, j],
                    gsem.at[buf],
                ).wait()

        def extract(i):
            buf = i % 2
            for g in range(K // _L):
                ids = idx_v[pl.ds(i * K + g * _L, _L)]
                for l in range(_L):
                    r = ids[l] & 7
                    j = g * _L + l
                    for c in range(D // _L):
                        out_v[buf, j // 2, pl.ds((j % 2) * D + c * _L, _L)] = (
                            tiles_v[buf, j, r, pl.ds(c * _L, _L)]
                        )

        def writeback_start(i):
            buf = i % 2
            pltpu.async_copy(
                out_v.at[buf],
                out_hbm.at[pl.ds(pl.multiple_of((base + i * K) // 2, 8), K // 2)],
                wsem.at[buf],
            )

        def writeback_wait(i):
            buf = i % 2
            pltpu.make_async_copy(
                out_v.at[buf],
                out_hbm.at[pl.ds(pl.multiple_of((base + i * K) // 2, 8), K // 2)],
                wsem.at[buf],
            ).wait()

        issue(0)

        def body(i, _):
            issue(i + 1)
            drain(i)
            lax.cond(i >= 2, lambda: writeback_wait(i - 2), lambda: None)
            extract(i)
            writeback_start(i)
            return 0

        lax.fori_loop(0, n_chunks - 1, body, 0)
        i_last = n_chunks - 1
        drain(i_last)
        writeback_wait(i_last - 2)
        extract(i_last)
        writeback_start(i_last)
        writeback_wait(i_last - 1)
        writeback_wait(i_last)

    return k


def kernel(node_ids, emb_weight):
    node_ids = node_ids.astype(jnp.int32)
    (B,) = node_ids.shape
    V, D = emb_weight.shape
    info = plsc.get_sparse_core_info()
    k = _build_gather(B, D, info.num_cores, info.num_subcores)
    out2 = k(node_ids, emb_weight)
    return out2.reshape(B, D)
